# baseline (speedup 1.0000x reference)
import functools

import jax
import jax.numpy as jnp
from jax import lax
from jax.experimental import pallas as pl
from jax.experimental.pallas import tpu as pltpu
from jax.experimental.pallas import tpu_sc as plsc

NC = 2
NS = 16
NW = NC * NS
CHUNK = 128
HB = 20


def _relu_tc(x):
    def body(x_ref, o_ref):
        o_ref[...] = jnp.maximum(x_ref[...], 0.0)

    return pl.pallas_call(
        body, out_shape=jax.ShapeDtypeStruct(x.shape, x.dtype)
    )(x)


def _sc_agg(rx, idxm, zeros, nt):
    np_, d = zeros.shape
    nb = nt // HB
    assert nt % HB == 0 and HB % 2 == 0
    rps = (np_ // NS) & ~7
    rps_last = np_ - (NS - 1) * rps
    assert rps > 0 and rps_last > 0

    mesh = plsc.VectorSubcoreMesh(core_axis_name="c", subcore_axis_name="s")

    @functools.partial(
        pl.kernel,
        out_type=jax.ShapeDtypeStruct((NC, np_, d), jnp.float32),
        mesh=mesh,
        scratch_types=[
            pltpu.VMEM((HB, 2, CHUNK), jnp.int32),
            pltpu.VMEM((HB, 2, CHUNK), jnp.int32),
            pltpu.VMEM((CHUNK, d), jnp.float32),
            pltpu.VMEM((CHUNK, d), jnp.float32),
            pltpu.SemaphoreType.DMA,
            pltpu.SemaphoreType.DMA,
            pltpu.SemaphoreType.DMA,
            pltpu.VMEM_SHARED((np_, d), jnp.float32),
        ],
    )
    def k(rx_hbm, idx_hbm, z_hbm, out_hbm,
          idxa, idxb, rows0, rows1, sem0, sem1, semi, acc):
        cid = lax.axis_index("c")
        sid = lax.axis_index("s")
        wid = sid * NC + cid

        row_base = pl.multiple_of(sid * rps, 8)

        @pl.when(sid < NS - 1)
        def _():
            pltpu.sync_copy(z_hbm.at[pl.ds(row_base, rps)],
                            acc.at[pl.ds(row_base, rps)])

        @pl.when(sid == NS - 1)
        def _():
            pltpu.sync_copy(z_hbm.at[pl.ds((NS - 1) * rps, rps_last)],
                            acc.at[pl.ds((NS - 1) * rps, rps_last)])

        w0 = wid * nt
        pltpu.sync_copy(idx_hbm.at[pl.ds(w0, HB)], idxa)

        plsc.subcore_barrier()

        ibufs = (idxa, idxb)
        rbufs = ((rows0, sem0), (rows1, sem1))

        pltpu.make_async_copy(rx_hbm.at[idxa.at[0, 0]], rows0, sem0).start()

        for b in range(nb):
            xb = ibufs[b % 2]
            xn = ibufs[1 - b % 2]
            if b + 1 < nb:
                pltpu.make_async_copy(
                    idx_hbm.at[pl.ds(w0 + (b + 1) * HB, HB)], xn, semi
                ).start()

            @pl.loop(0, HB, step=2)
            def _(i):
                for t in range(2):
                    j = i + t
                    rows_c, sem_c = rbufs[t]
                    rows_n, sem_n = rbufs[1 - t]

                    @pl.when(j + 1 < HB)
                    def _():
                        pltpu.make_async_copy(
                            rx_hbm.at[xb.at[j + 1, 0]], rows_n, sem_n
                        ).start()

                    pltpu.make_async_copy(
                        rx_hbm.at[xb.at[j, 0]], rows_c, sem_c).wait()
                    pltpu.sync_copy(rows_c, acc.at[xb.at[j, 1]], add=True)

            if b + 1 < nb:
                pltpu.make_async_copy(
                    idx_hbm.at[pl.ds(w0 + (b + 1) * HB, HB)], xn, semi
                ).wait()
                pltpu.make_async_copy(
                    rx_hbm.at[xn.at[0, 0]], rows0, sem0).start()

        plsc.subcore_barrier()

        @pl.when(sid < NS - 1)
        def _():
            pltpu.sync_copy(acc.at[pl.ds(row_base, rps)],
                            out_hbm.at[cid].at[pl.ds(row_base, rps)])

        @pl.when(sid == NS - 1)
        def _():
            pltpu.sync_copy(acc.at[pl.ds((NS - 1) * rps, rps_last)],
                            out_hbm.at[cid].at[pl.ds((NS - 1) * rps, rps_last)])

    return k(rx, idxm, zeros)


def _mlp_tc(x, p0, p1, W1, b1, W2, b2):
    n, d = x.shape
    bn = 1000
    assert n % bn == 0

    def body(x_ref, p0_ref, p1_ref, w1_ref, b1_ref, w2_ref, b2_ref, o_ref):
        h = x_ref[...] + p0_ref[...] + p1_ref[...]
        h = jnp.dot(h, w1_ref[...], preferred_element_type=jnp.float32)
        h = jnp.maximum(h + b1_ref[...], 0.0)
        h = jnp.dot(h, w2_ref[...], preferred_element_type=jnp.float32)
        o_ref[...] = jnp.maximum(h + b2_ref[...], 0.0)

    row_spec = pl.BlockSpec((bn, d), lambda i: (i, 0))
    full_spec = pl.BlockSpec((d, d), lambda i: (0, 0))
    bias_spec = pl.BlockSpec((1, d), lambda i: (0, 0))
    return pl.pallas_call(
        body,
        grid=(n // bn,),
        in_specs=[row_spec, row_spec, row_spec, full_spec, bias_spec,
                  full_spec, bias_spec],
        out_specs=row_spec,
        out_shape=jax.ShapeDtypeStruct((n, d), jnp.float32),
    )(x, p0, p1, W1, b1, W2, b2)


def kernel(x, edge_index, W1, b1, W2, b2):
    n, d = x.shape
    src = edge_index[0]
    dst = edge_index[1]
    e = src.shape[0]

    num_chunks = -(-e // CHUNK)
    nt = -(-num_chunks // NW)
    nt = -(-nt // HB) * HB
    e_pad = NW * nt * CHUNK
    pad = e_pad - e
    srcm = jnp.concatenate(
        [src, jnp.zeros((pad,), jnp.int32)]).reshape(NW * nt, CHUNK)
    dstm = jnp.concatenate(
        [dst, jnp.full((pad,), n, jnp.int32)]).reshape(NW * nt, CHUNK)
    idxm = jnp.stack([srcm, dstm], axis=1)

    rx = _relu_tc(x)
    zeros = jnp.zeros((n + 8, d), jnp.float32)
    partials = _sc_agg(rx, idxm, zeros, nt)
    return _mlp_tc(x, partials[0, :n], partials[1, :n], W1,
                   b1.reshape(1, d), W2, b2.reshape(1, d))


# speedup vs baseline: 2.5154x; 2.5154x over previous
import functools

import jax
import jax.numpy as jnp
from jax import lax
from jax.experimental import pallas as pl
from jax.experimental.pallas import tpu as pltpu
from jax.experimental.pallas import tpu_sc as plsc

NC = 2
NS = 16
NW = NC * NS
CHUNK = 128


def _relu_tc(x):
    def body(x_ref, o_ref):
        o_ref[...] = jnp.maximum(x_ref[...], 0.0)

    return pl.pallas_call(
        body, out_shape=jax.ShapeDtypeStruct(x.shape, x.dtype)
    )(x)


def _sc_agg(rx, src, dst, zeros):
    n, d = rx.shape
    e = src.shape[0]
    assert e % CHUNK == 0
    num_chunks = e // CHUNK
    rps = (n // NS) & ~7
    rps_last = n - (NS - 1) * rps
    assert rps > 0 and rps_last > 0

    mesh = plsc.VectorSubcoreMesh(core_axis_name="c", subcore_axis_name="s")

    @functools.partial(
        pl.kernel,
        out_type=jax.ShapeDtypeStruct((NC, n, d), jnp.float32),
        mesh=mesh,
        scratch_types=[
            pltpu.VMEM((CHUNK,), jnp.int32),
            pltpu.VMEM((CHUNK,), jnp.int32),
            pltpu.VMEM((CHUNK,), jnp.int32),
            pltpu.VMEM((CHUNK,), jnp.int32),
            pltpu.VMEM((CHUNK, d), jnp.float32),
            pltpu.VMEM((CHUNK, d), jnp.float32),
            pltpu.SemaphoreType.DMA,
            pltpu.SemaphoreType.DMA,
            pltpu.VMEM_SHARED((n, d), jnp.float32),
        ],
    )
    def k(rx_hbm, src_hbm, dst_hbm, z_hbm, out_hbm,
          sidx0, sidx1, didx0, didx1, rows0, rows1, sem0, sem1, acc):
        cid = lax.axis_index("c")
        sid = lax.axis_index("s")
        wid = sid * NC + cid

        row_base = pl.multiple_of(sid * rps, 8)

        @pl.when(sid < NS - 1)
        def _():
            pltpu.sync_copy(z_hbm.at[pl.ds(row_base, rps)],
                            acc.at[pl.ds(row_base, rps)])

        @pl.when(sid == NS - 1)
        def _():
            pltpu.sync_copy(z_hbm.at[pl.ds((NS - 1) * rps, rps_last)],
                            acc.at[pl.ds((NS - 1) * rps, rps_last)])

        plsc.subcore_barrier()

        nt = (num_chunks - wid + NW - 1) // NW

        bufs = ((sidx0, didx0, rows0, sem0), (sidx1, didx1, rows1, sem1))

        @pl.when(nt > 0)
        def _():
            base = pl.multiple_of(wid * CHUNK, CHUNK)
            pltpu.sync_copy(src_hbm.at[pl.ds(base, CHUNK)], sidx0)
            pltpu.sync_copy(dst_hbm.at[pl.ds(base, CHUNK)], didx0)
            pltpu.make_async_copy(rx_hbm.at[sidx0], rows0, sem0).start()

        @pl.loop(0, (nt + 1) // 2)
        def _(p):
            for t in range(2):
                j = 2 * p + t
                sidx_c, didx_c, rows_c, sem_c = bufs[t]
                sidx_n, didx_n, rows_n, sem_n = bufs[1 - t]

                @pl.when(j < nt)
                def _():
                    @pl.when(j + 1 < nt)
                    def _():
                        base = pl.multiple_of(
                            (wid + (j + 1) * NW) * CHUNK, CHUNK)
                        pltpu.sync_copy(src_hbm.at[pl.ds(base, CHUNK)], sidx_n)
                        pltpu.sync_copy(dst_hbm.at[pl.ds(base, CHUNK)], didx_n)
                        pltpu.make_async_copy(
                            rx_hbm.at[sidx_n], rows_n, sem_n).start()

                    pltpu.make_async_copy(
                        rx_hbm.at[sidx_c], rows_c, sem_c).wait()
                    pltpu.sync_copy(rows_c, acc.at[didx_c], add=True)

        plsc.subcore_barrier()

        @pl.when(sid < NS - 1)
        def _():
            pltpu.sync_copy(acc.at[pl.ds(row_base, rps)],
                            out_hbm.at[cid].at[pl.ds(row_base, rps)])

        @pl.when(sid == NS - 1)
        def _():
            pltpu.sync_copy(acc.at[pl.ds((NS - 1) * rps, rps_last)],
                            out_hbm.at[cid].at[pl.ds((NS - 1) * rps, rps_last)])

    return k(rx, src, dst, zeros)


def _mlp_tc(x, p0, p1, W1, b1, W2, b2):
    n, d = x.shape
    bn = 1000
    assert n % bn == 0

    def body(x_ref, p0_ref, p1_ref, w1_ref, b1_ref, w2_ref, b2_ref, o_ref):
        h = x_ref[...] + p0_ref[...] + p1_ref[...]
        h = jnp.dot(h, w1_ref[...], preferred_element_type=jnp.float32)
        h = jnp.maximum(h + b1_ref[...], 0.0)
        h = jnp.dot(h, w2_ref[...], preferred_element_type=jnp.float32)
        o_ref[...] = jnp.maximum(h + b2_ref[...], 0.0)

    row_spec = pl.BlockSpec((bn, d), lambda i: (i, 0))
    full_spec = pl.BlockSpec((d, d), lambda i: (0, 0))
    bias_spec = pl.BlockSpec((1, d), lambda i: (0, 0))
    return pl.pallas_call(
        body,
        grid=(n // bn,),
        in_specs=[row_spec, row_spec, row_spec, full_spec, bias_spec,
                  full_spec, bias_spec],
        out_specs=row_spec,
        out_shape=jax.ShapeDtypeStruct((n, d), jnp.float32),
    )(x, p0, p1, W1, b1, W2, b2)


def kernel(x, edge_index, W1, b1, W2, b2):
    n, d = x.shape
    src = edge_index[0]
    dst = edge_index[1]
    rx = _relu_tc(x)
    zeros = jnp.zeros((n, d), jnp.float32)
    partials = _sc_agg(rx, src, dst, zeros)
    return _mlp_tc(x, partials[0], partials[1], W1,
                   b1.reshape(1, d), W2, b2.reshape(1, d))


# speedup vs baseline: 3.1930x; 1.2694x over previous
import functools

import jax
import jax.numpy as jnp
from jax import lax
from jax.experimental import pallas as pl
from jax.experimental.pallas import tpu as pltpu
from jax.experimental.pallas import tpu_sc as plsc

NC = 2
NS = 16
NW = NC * NS
CHUNK = 128
NB = 3


def _relu_tc(x):
    def body(x_ref, o_ref):
        o_ref[...] = jnp.maximum(x_ref[...], 0.0)

    return pl.pallas_call(
        body, out_shape=jax.ShapeDtypeStruct(x.shape, x.dtype)
    )(x)


def _sc_agg(x, rx, src, dst, zeros):
    n, d = rx.shape
    e = src.shape[0]
    assert e % CHUNK == 0
    num_chunks = e // CHUNK
    rps = (n // NS) & ~7
    rps_last = n - (NS - 1) * rps
    assert rps > 0 and rps_last > 0

    mesh = plsc.VectorSubcoreMesh(core_axis_name="c", subcore_axis_name="s")

    @functools.partial(
        pl.kernel,
        out_type=jax.ShapeDtypeStruct((NC, n, d), jnp.float32),
        mesh=mesh,
        scratch_types=[
            pltpu.VMEM((CHUNK,), jnp.int32),
            pltpu.VMEM((CHUNK,), jnp.int32),
            pltpu.VMEM((CHUNK,), jnp.int32),
            pltpu.VMEM((CHUNK,), jnp.int32),
            pltpu.VMEM((CHUNK,), jnp.int32),
            pltpu.VMEM((CHUNK,), jnp.int32),
            pltpu.VMEM((CHUNK, d), jnp.float32),
            pltpu.VMEM((CHUNK, d), jnp.float32),
            pltpu.VMEM((CHUNK, d), jnp.float32),
            pltpu.SemaphoreType.DMA,
            pltpu.SemaphoreType.DMA,
            pltpu.SemaphoreType.DMA,
            pltpu.SemaphoreType.DMA,
            pltpu.SemaphoreType.DMA,
            pltpu.SemaphoreType.DMA,
            pltpu.SemaphoreType.DMA,
            pltpu.VMEM_SHARED((n, d), jnp.float32),
        ],
    )
    def k(x_hbm, rx_hbm, src_hbm, dst_hbm, z_hbm, out_hbm,
          s0, s1, s2, d0, d1, d2, r0, r1, r2,
          g0, g1, g2, c0, c1, c2, semi, acc):
        cid = lax.axis_index("c")
        sid = lax.axis_index("s")
        wid = sid * NC + cid

        row_base = pl.multiple_of(sid * rps, 8)
        sidx = (s0, s1, s2)
        didx = (d0, d1, d2)
        rows = (r0, r1, r2)
        gsem = (g0, g1, g2)
        csem = (c0, c1, c2)

        def rowwise_copy(src_ref, dst_ref):
            @pl.when(sid < NS - 1)
            def _():
                pltpu.sync_copy(src_ref.at[pl.ds(row_base, rps)],
                                dst_ref.at[pl.ds(row_base, rps)])

            @pl.when(sid == NS - 1)
            def _():
                pltpu.sync_copy(src_ref.at[pl.ds((NS - 1) * rps, rps_last)],
                                dst_ref.at[pl.ds((NS - 1) * rps, rps_last)])

        @pl.when(cid == 0)
        def _():
            rowwise_copy(x_hbm, acc)

        @pl.when(cid != 0)
        def _():
            rowwise_copy(z_hbm, acc)

        plsc.subcore_barrier()

        nt = (num_chunks - wid + NW - 1) // NW

        def idx_base(j):
            return pl.multiple_of((wid + j * NW) * CHUNK, CHUNK)

        def idx_copies(j, sidx_b, didx_b):
            base = idx_base(j)
            return (pltpu.make_async_copy(
                        src_hbm.at[pl.ds(base, CHUNK)], sidx_b, semi),
                    pltpu.make_async_copy(
                        dst_hbm.at[pl.ds(base, CHUNK)], didx_b, semi))

        @pl.when(nt > 0)
        def _():
            base = idx_base(0)
            pltpu.sync_copy(src_hbm.at[pl.ds(base, CHUNK)], s0)
            pltpu.sync_copy(dst_hbm.at[pl.ds(base, CHUNK)], d0)
            pltpu.make_async_copy(rx_hbm.at[s0], r0, g0).start()

            @pl.when(nt > 1)
            def _():
                for cpy in idx_copies(1, s1, d1):
                    cpy.start()

        @pl.loop(0, (nt + NB - 1) // NB)
        def _(p):
            for t in range(NB):
                j = NB * p + t
                cs = t
                nx = (t + 1) % NB
                fs = (t + 2) % NB

                @pl.when(j < nt)
                def _():
                    @pl.when(j + 1 < nt)
                    def _():
                        for cpy in idx_copies(j + 1, sidx[nx], didx[nx]):
                            cpy.wait()
                        pltpu.make_async_copy(
                            rx_hbm.at[sidx[nx]], rows[nx], gsem[nx]).start()

                    pltpu.make_async_copy(
                        rx_hbm.at[sidx[cs]], rows[cs], gsem[cs]).wait()
                    pltpu.make_async_copy(
                        rows[cs], acc.at[didx[cs]], csem[cs]).start(add=True)

                    @pl.when(j >= 1)
                    def _():
                        pltpu.make_async_copy(
                            rows[fs], acc.at[didx[fs]], csem[fs]).wait()

                    @pl.when(j + 2 < nt)
                    def _():
                        for cpy in idx_copies(j + 2, sidx[fs], didx[fs]):
                            cpy.start()

        for t in range(NB):
            @pl.when((nt > 0) & ((nt - 1) % NB == t))
            def _():
                pltpu.make_async_copy(
                    rows[t], acc.at[didx[t]], csem[t]).wait()

        plsc.subcore_barrier()

        rowwise_copy(acc, out_hbm.at[cid])

    return k(x, rx, src, dst, zeros)


def _mlp_tc(p0, p1, W1, b1, W2, b2):
    n, d = p0.shape
    bn = 1000
    assert n % bn == 0

    def body(p0_ref, p1_ref, w1_ref, b1_ref, w2_ref, b2_ref, o_ref):
        h = p0_ref[...] + p1_ref[...]
        h = jnp.dot(h, w1_ref[...], preferred_element_type=jnp.float32)
        h = jnp.maximum(h + b1_ref[...], 0.0)
        h = jnp.dot(h, w2_ref[...], preferred_element_type=jnp.float32)
        o_ref[...] = jnp.maximum(h + b2_ref[...], 0.0)

    row_spec = pl.BlockSpec((bn, d), lambda i: (i, 0))
    full_spec = pl.BlockSpec((d, d), lambda i: (0, 0))
    bias_spec = pl.BlockSpec((1, d), lambda i: (0, 0))
    return pl.pallas_call(
        body,
        grid=(n // bn,),
        in_specs=[row_spec, row_spec, full_spec, bias_spec,
                  full_spec, bias_spec],
        out_specs=row_spec,
        out_shape=jax.ShapeDtypeStruct((n, d), jnp.float32),
    )(p0, p1, W1, b1, W2, b2)


def kernel(x, edge_index, W1, b1, W2, b2):
    n, d = x.shape
    src = edge_index[0]
    dst = edge_index[1]
    rx = _relu_tc(x)
    zeros = jnp.zeros((n, d), jnp.float32)
    partials = _sc_agg(x, rx, src, dst, zeros)
    return _mlp_tc(partials[0], partials[1], W1,
                   b1.reshape(1, d), W2, b2.reshape(1, d))


# speedup vs baseline: 3.3944x; 1.0631x over previous
import functools

import jax
import jax.numpy as jnp
from jax import lax
from jax.experimental import pallas as pl
from jax.experimental.pallas import tpu as pltpu
from jax.experimental.pallas import tpu_sc as plsc

NC = 2
NS = 16
NW = NC * NS
CHUNK = 80
NB = 4


def _relu_tc(x):
    def body(x_ref, o_ref):
        o_ref[...] = jnp.maximum(x_ref[...], 0.0)

    return pl.pallas_call(
        body, out_shape=jax.ShapeDtypeStruct(x.shape, x.dtype)
    )(x)


def _sc_agg(x, rx, src, dst):
    n, d = rx.shape
    e = src.shape[0]
    assert e % CHUNK == 0
    num_chunks = e // CHUNK
    rps = (n // NS) & ~7
    rps_last = n - (NS - 1) * rps
    assert rps > 0 and rps_last > 0

    mesh = plsc.VectorSubcoreMesh(core_axis_name="c", subcore_axis_name="s")

    @functools.partial(
        pl.kernel,
        out_type=jax.ShapeDtypeStruct((NC, n, d), jnp.float32),
        mesh=mesh,
        scratch_types=[
            pltpu.VMEM((CHUNK,), jnp.int32),
            pltpu.VMEM((CHUNK,), jnp.int32),
            pltpu.VMEM((CHUNK,), jnp.int32),
            pltpu.VMEM((CHUNK,), jnp.int32),
            pltpu.VMEM((CHUNK,), jnp.int32),
            pltpu.VMEM((CHUNK,), jnp.int32),
            pltpu.VMEM((CHUNK,), jnp.int32),
            pltpu.VMEM((CHUNK,), jnp.int32),
            pltpu.VMEM((CHUNK, d), jnp.float32),
            pltpu.VMEM((CHUNK, d), jnp.float32),
            pltpu.VMEM((CHUNK, d), jnp.float32),
            pltpu.VMEM((CHUNK, d), jnp.float32),
            pltpu.SemaphoreType.DMA,
            pltpu.SemaphoreType.DMA,
            pltpu.SemaphoreType.DMA,
            pltpu.SemaphoreType.DMA,
            pltpu.SemaphoreType.DMA,
            pltpu.SemaphoreType.DMA,
            pltpu.SemaphoreType.DMA,
            pltpu.SemaphoreType.DMA,
            pltpu.SemaphoreType.DMA,
            pltpu.VMEM_SHARED((n, d), jnp.float32),
        ],
    )
    def k(x_hbm, src_hbm, dst_hbm, rx_hbm, out_hbm,
          s0, s1, s2, s3, d0, d1, d2, d3, r0, r1, r2, r3,
          g0, g1, g2, g3, c0, c1, c2, c3, semi, acc):
        cid = lax.axis_index("c")
        sid = lax.axis_index("s")
        wid = sid * NC + cid

        row_base = pl.multiple_of(sid * rps, 8)
        sidx = (s0, s1, s2, s3)
        didx = (d0, d1, d2, d3)
        rows = (r0, r1, r2, r3)
        gsem = (g0, g1, g2, g3)
        csem = (c0, c1, c2, c3)

        def blockwise(total, fn):
            off = 0
            while off < total:
                blk = min(CHUNK, total - off)
                fn(off, blk)
                off += blk

        @pl.when(cid == 0)
        def _():
            @pl.when(sid < NS - 1)
            def _():
                pltpu.sync_copy(x_hbm.at[pl.ds(row_base, rps)],
                                acc.at[pl.ds(row_base, rps)])

            @pl.when(sid == NS - 1)
            def _():
                pltpu.sync_copy(x_hbm.at[pl.ds((NS - 1) * rps, rps_last)],
                                acc.at[pl.ds((NS - 1) * rps, rps_last)])

        @pl.when(cid != 0)
        def _():
            @pl.loop(0, CHUNK)
            def _(i):
                for col in range(d // 16):
                    r0[i, pl.ds(col * 16, 16)] = jnp.zeros((16,), jnp.float32)

            @pl.when(sid < NS - 1)
            def _():
                blockwise(rps, lambda off, blk: pltpu.sync_copy(
                    r0.at[pl.ds(0, blk)],
                    acc.at[pl.ds(sid * rps + off, blk)]))

            @pl.when(sid == NS - 1)
            def _():
                blockwise(rps_last, lambda off, blk: pltpu.sync_copy(
                    r0.at[pl.ds(0, blk)],
                    acc.at[pl.ds((NS - 1) * rps + off, blk)]))

        plsc.subcore_barrier()

        nt = (num_chunks - wid + NW - 1) // NW

        def idx_base(j):
            return pl.multiple_of((wid + j * NW) * CHUNK, 8)

        def idx_copies(j, sidx_b, didx_b):
            base = idx_base(j)
            return (pltpu.make_async_copy(
                        src_hbm.at[pl.ds(base, CHUNK)], sidx_b, semi),
                    pltpu.make_async_copy(
                        dst_hbm.at[pl.ds(base, CHUNK)], didx_b, semi))

        @pl.when(nt > 0)
        def _():
            base = idx_base(0)
            pltpu.sync_copy(src_hbm.at[pl.ds(base, CHUNK)], s0)
            pltpu.sync_copy(dst_hbm.at[pl.ds(base, CHUNK)], d0)
            pltpu.make_async_copy(rx_hbm.at[s0], r0, g0).start()

            @pl.when(nt > 1)
            def _():
                base1 = idx_base(1)
                pltpu.sync_copy(src_hbm.at[pl.ds(base1, CHUNK)], s1)
                pltpu.sync_copy(dst_hbm.at[pl.ds(base1, CHUNK)], d1)
                pltpu.make_async_copy(rx_hbm.at[s1], r1, g1).start()

                @pl.when(nt > 2)
                def _():
                    for cpy in idx_copies(2, s2, d2):
                        cpy.start()

        @pl.loop(0, (nt + NB - 1) // NB)
        def _(p):
            for t in range(NB):
                j = NB * p + t
                cs = t
                g2s = (t + 2) % NB
                ps = (t + 3) % NB

                @pl.when(j < nt)
                def _():
                    @pl.when(j + 2 < nt)
                    def _():
                        for cpy in idx_copies(j + 2, sidx[g2s], didx[g2s]):
                            cpy.wait()
                        pltpu.make_async_copy(
                            rx_hbm.at[sidx[g2s]], rows[g2s], gsem[g2s]).start()

                    pltpu.make_async_copy(
                        rx_hbm.at[sidx[cs]], rows[cs], gsem[cs]).wait()
                    pltpu.make_async_copy(
                        rows[cs], acc.at[didx[cs]], csem[cs]).start(add=True)

                    @pl.when(j >= 1)
                    def _():
                        pltpu.make_async_copy(
                            rows[ps], acc.at[didx[ps]], csem[ps]).wait()

                    @pl.when(j + 3 < nt)
                    def _():
                        for cpy in idx_copies(j + 3, sidx[ps], didx[ps]):
                            cpy.start()

        for t in range(NB):
            @pl.when((nt > 0) & ((nt - 1) % NB == t))
            def _():
                pltpu.make_async_copy(
                    rows[t], acc.at[didx[t]], csem[t]).wait()

        plsc.subcore_barrier()

        @pl.when(sid < NS - 1)
        def _():
            pltpu.sync_copy(acc.at[pl.ds(row_base, rps)],
                            out_hbm.at[cid].at[pl.ds(row_base, rps)])

        @pl.when(sid == NS - 1)
        def _():
            pltpu.sync_copy(acc.at[pl.ds((NS - 1) * rps, rps_last)],
                            out_hbm.at[cid].at[pl.ds((NS - 1) * rps, rps_last)])

    return k(x, src, dst, rx)


def _mlp_tc(p0, p1, W1, b1, W2, b2):
    n, d = p0.shape
    bn = 1000
    assert n % bn == 0

    def body(p0_ref, p1_ref, w1_ref, b1_ref, w2_ref, b2_ref, o_ref):
        h = p0_ref[...] + p1_ref[...]
        h = jnp.dot(h, w1_ref[...], preferred_element_type=jnp.float32)
        h = jnp.maximum(h + b1_ref[...], 0.0)
        h = jnp.dot(h, w2_ref[...], preferred_element_type=jnp.float32)
        o_ref[...] = jnp.maximum(h + b2_ref[...], 0.0)

    row_spec = pl.BlockSpec((bn, d), lambda i: (i, 0))
    full_spec = pl.BlockSpec((d, d), lambda i: (0, 0))
    bias_spec = pl.BlockSpec((1, d), lambda i: (0, 0))
    return pl.pallas_call(
        body,
        grid=(n // bn,),
        in_specs=[row_spec, row_spec, full_spec, bias_spec,
                  full_spec, bias_spec],
        out_specs=row_spec,
        out_shape=jax.ShapeDtypeStruct((n, d), jnp.float32),
    )(p0, p1, W1, b1, W2, b2)


def kernel(x, edge_index, W1, b1, W2, b2):
    n, d = x.shape
    src = edge_index[0]
    dst = edge_index[1]
    rx = _relu_tc(x)
    partials = _sc_agg(x, rx, src, dst)
    return _mlp_tc(partials[0], partials[1], W1,
                   b1.reshape(1, d), W2, b2.reshape(1, d))


# speedup vs baseline: 3.4093x; 1.0044x over previous
import functools

import jax
import jax.numpy as jnp
from jax import lax
from jax.experimental import pallas as pl
from jax.experimental.pallas import tpu as pltpu
from jax.experimental.pallas import tpu_sc as plsc

NC = 2
NS = 16
NW = NC * NS
CHUNK = 64
NB = 5
GA = 3


def _relu_tc(x):
    def body(x_ref, o_ref):
        o_ref[...] = jnp.maximum(x_ref[...], 0.0)

    return pl.pallas_call(
        body, out_shape=jax.ShapeDtypeStruct(x.shape, x.dtype)
    )(x)


def _sc_agg(x, rx, src, dst):
    n, d = rx.shape
    e = src.shape[0]
    assert e % CHUNK == 0
    num_chunks = e // CHUNK
    rps = (n // NS) & ~7
    rps_last = n - (NS - 1) * rps
    assert rps > 0 and rps_last > 0

    mesh = plsc.VectorSubcoreMesh(core_axis_name="c", subcore_axis_name="s")

    @functools.partial(
        pl.kernel,
        out_type=jax.ShapeDtypeStruct((NC, n, d), jnp.float32),
        mesh=mesh,
        scratch_types=(
            [pltpu.VMEM((CHUNK,), jnp.int32)] * NB
            + [pltpu.VMEM((CHUNK,), jnp.int32)] * NB
            + [pltpu.VMEM((CHUNK, d), jnp.float32)] * NB
            + [pltpu.SemaphoreType.DMA] * NB
            + [pltpu.SemaphoreType.DMA] * NB
            + [pltpu.SemaphoreType.DMA]
            + [pltpu.VMEM_SHARED((n, d), jnp.float32)]
        ),
    )
    def k(x_hbm, src_hbm, dst_hbm, rx_hbm, out_hbm, *refs):
        sidx = refs[0:NB]
        didx = refs[NB:2 * NB]
        rows = refs[2 * NB:3 * NB]
        gsem = refs[3 * NB:4 * NB]
        csem = refs[4 * NB:5 * NB]
        semi = refs[5 * NB]
        acc = refs[5 * NB + 1]

        cid = lax.axis_index("c")
        sid = lax.axis_index("s")
        wid = sid * NC + cid

        row_base = pl.multiple_of(sid * rps, 8)
        r0 = rows[0]

        def blockwise(total, fn):
            off = 0
            while off < total:
                blk = min(CHUNK, total - off)
                fn(off, blk)
                off += blk

        @pl.when(cid == 0)
        def _():
            @pl.when(sid < NS - 1)
            def _():
                pltpu.sync_copy(x_hbm.at[pl.ds(row_base, rps)],
                                acc.at[pl.ds(row_base, rps)])

            @pl.when(sid == NS - 1)
            def _():
                pltpu.sync_copy(x_hbm.at[pl.ds((NS - 1) * rps, rps_last)],
                                acc.at[pl.ds((NS - 1) * rps, rps_last)])

        @pl.when(cid != 0)
        def _():
            @pl.loop(0, CHUNK)
            def _(i):
                for col in range(d // 16):
                    r0[i, pl.ds(col * 16, 16)] = jnp.zeros((16,), jnp.float32)

            @pl.when(sid < NS - 1)
            def _():
                blockwise(rps, lambda off, blk: pltpu.sync_copy(
                    r0.at[pl.ds(0, blk)],
                    acc.at[pl.ds(sid * rps + off, blk)]))

            @pl.when(sid == NS - 1)
            def _():
                blockwise(rps_last, lambda off, blk: pltpu.sync_copy(
                    r0.at[pl.ds(0, blk)],
                    acc.at[pl.ds((NS - 1) * rps + off, blk)]))

        plsc.subcore_barrier()

        nt = (num_chunks - wid + NW - 1) // NW

        def idx_base(j):
            return pl.multiple_of((wid + j * NW) * CHUNK, 8)

        def idx_copies(j, sidx_b, didx_b):
            base = idx_base(j)
            return (pltpu.make_async_copy(
                        src_hbm.at[pl.ds(base, CHUNK)], sidx_b, semi),
                    pltpu.make_async_copy(
                        dst_hbm.at[pl.ds(base, CHUNK)], didx_b, semi))

        def prime(j):
            @pl.when(nt > j)
            def _():
                base = idx_base(j)
                pltpu.sync_copy(src_hbm.at[pl.ds(base, CHUNK)], sidx[j])
                pltpu.sync_copy(dst_hbm.at[pl.ds(base, CHUNK)], didx[j])
                pltpu.make_async_copy(rx_hbm.at[sidx[j]], rows[j],
                                      gsem[j]).start()
                if j + 1 < GA:
                    prime(j + 1)
                else:
                    @pl.when(nt > GA)
                    def _():
                        for cpy in idx_copies(GA, sidx[GA], didx[GA]):
                            cpy.start()

        prime(0)

        @pl.loop(0, (nt + NB - 1) // NB)
        def _(p):
            for t in range(NB):
                j = NB * p + t
                cs = t
                gs = (t + GA) % NB
                ps = (t + NB - 1) % NB

                @pl.when(j < nt)
                def _():
                    @pl.when(j + GA < nt)
                    def _():
                        for cpy in idx_copies(j + GA, sidx[gs], didx[gs]):
                            cpy.wait()
                        pltpu.make_async_copy(
                            rx_hbm.at[sidx[gs]], rows[gs], gsem[gs]).start()

                    pltpu.make_async_copy(
                        rx_hbm.at[sidx[cs]], rows[cs], gsem[cs]).wait()
                    pltpu.make_async_copy(
                        rows[cs], acc.at[didx[cs]], csem[cs]).start(add=True)

                    @pl.when(j >= 1)
                    def _():
                        pltpu.make_async_copy(
                            rows[ps], acc.at[didx[ps]], csem[ps]).wait()

                    @pl.when(j + NB - 1 < nt)
                    def _():
                        for cpy in idx_copies(j + NB - 1, sidx[ps], didx[ps]):
                            cpy.start()

        for t in range(NB):
            @pl.when((nt > 0) & ((nt - 1) % NB == t))
            def _():
                pltpu.make_async_copy(
                    rows[t], acc.at[didx[t]], csem[t]).wait()

        plsc.subcore_barrier()

        @pl.when(sid < NS - 1)
        def _():
            pltpu.sync_copy(acc.at[pl.ds(row_base, rps)],
                            out_hbm.at[cid].at[pl.ds(row_base, rps)])

        @pl.when(sid == NS - 1)
        def _():
            pltpu.sync_copy(acc.at[pl.ds((NS - 1) * rps, rps_last)],
                            out_hbm.at[cid].at[pl.ds((NS - 1) * rps, rps_last)])

    return k(x, src, dst, rx)


def _mlp_tc(p0, p1, W1, b1, W2, b2):
    n, d = p0.shape
    bn = 2000
    assert n % bn == 0

    def body(p0_ref, p1_ref, w1_ref, b1_ref, w2_ref, b2_ref, o_ref):
        h = p0_ref[...] + p1_ref[...]
        h = jnp.dot(h, w1_ref[...], preferred_element_type=jnp.float32)
        h = jnp.maximum(h + b1_ref[...], 0.0)
        h = jnp.dot(h, w2_ref[...], preferred_element_type=jnp.float32)
        o_ref[...] = jnp.maximum(h + b2_ref[...], 0.0)

    row_spec = pl.BlockSpec((bn, d), lambda i: (i, 0))
    full_spec = pl.BlockSpec((d, d), lambda i: (0, 0))
    bias_spec = pl.BlockSpec((1, d), lambda i: (0, 0))
    return pl.pallas_call(
        body,
        grid=(n // bn,),
        in_specs=[row_spec, row_spec, full_spec, bias_spec,
                  full_spec, bias_spec],
        out_specs=row_spec,
        out_shape=jax.ShapeDtypeStruct((n, d), jnp.float32),
    )(p0, p1, W1, b1, W2, b2)


def kernel(x, edge_index, W1, b1, W2, b2):
    n, d = x.shape
    src = edge_index[0]
    dst = edge_index[1]
    rx = _relu_tc(x)
    partials = _sc_agg(x, rx, src, dst)
    return _mlp_tc(partials[0], partials[1], W1,
                   b1.reshape(1, d), W2, b2.reshape(1, d))
